# Initial kernel scaffold; baseline (speedup 1.0000x reference)
import jax
import jax.numpy as jnp
from jax.experimental import pallas as pl


def kernel(hidden_states, W_router, W1, b1, W2, b2):
    raise NotImplementedError("write your pallas kernel here")


# speedup vs baseline: 7.7275x; 7.7275x over previous
import functools

import jax
import jax.numpy as jnp
from jax import lax
from jax.experimental import pallas as pl
from jax.experimental.pallas import tpu as pltpu
from jax.experimental.pallas import tpu_sc as plsc

_SC_CORES = 2
_SC_SUBCORES = 16
_SC_WORKERS = _SC_CORES * _SC_SUBCORES
_SC_CHUNK = 32

_TM = 128


def _router(x, w_router):
    t, _ = x.shape

    def body(x_ref, wr_ref, gate_ref, eid_ref):
        logits = jnp.dot(x_ref[...], wr_ref[...],
                         preferred_element_type=jnp.float32)
        m = jnp.max(logits, axis=-1, keepdims=True)
        ssum = jnp.sum(jnp.exp(logits - m), axis=-1, keepdims=True)
        gate_ref[...] = 1.0 / ssum
        eid_ref[...] = jnp.argmax(logits, axis=-1, keepdims=True).astype(
            jnp.int32)

    return pl.pallas_call(
        body,
        out_shape=(jax.ShapeDtypeStruct((t, 1), jnp.float32),
                   jax.ShapeDtypeStruct((t, 1), jnp.int32)),
    )(x, w_router)


def _route_metadata(expert, num_experts, tm, num_slots):
    t = expert.shape[0]
    perm = jnp.argsort(expert).astype(jnp.int32)
    counts = jnp.zeros((num_experts,), jnp.int32).at[expert].add(1)
    go = jnp.concatenate(
        [jnp.zeros((1,), jnp.int32), jnp.cumsum(counts, dtype=jnp.int32)])
    first_tile = go[:num_experts] // tm
    last_tile = jnp.maximum(go[1:], 1).astype(jnp.int32)
    last_tile = (last_tile - 1) // tm
    span = jnp.where(counts > 0, last_tile - first_tile + 1, 0).astype(jnp.int32)
    cum = jnp.cumsum(span, dtype=jnp.int32)
    slot_start = cum - span
    total = cum[num_experts - 1]

    i = jnp.arange(num_slots, dtype=jnp.int32)
    g_i = jnp.searchsorted(cum, i, side="right").astype(jnp.int32)
    g_i = jnp.minimum(g_i, num_experts - 1)
    tile_i = first_tile[g_i] + (i - slot_start[g_i])
    valid = i < total
    g_last = jnp.searchsorted(cum, total - 1, side="right").astype(jnp.int32)
    g_last = jnp.minimum(g_last, num_experts - 1)
    tile_last = last_tile[g_last]
    g_meta = jnp.where(valid, g_i, g_last).astype(jnp.int32)
    t_meta = jnp.where(valid, tile_i, tile_last).astype(jnp.int32)
    lo = jnp.where(valid, go[g_meta], 0).astype(jnp.int32)
    hi = jnp.where(valid, go[jnp.minimum(g_meta + 1, num_experts)], 0)
    hi = jnp.where(valid, hi, 0).astype(jnp.int32)
    prev_tile = jnp.concatenate([jnp.full((1,), -1, jnp.int32), t_meta[:-1]])
    first = (valid & (t_meta != prev_tile)).astype(jnp.int32)
    return perm, g_meta, t_meta, lo, hi, first


def _sc_gather(x, perm):
    t, d = x.shape
    rows_per_w = t // _SC_WORKERS
    n_ch = rows_per_w // _SC_CHUNK
    mesh = plsc.VectorSubcoreMesh(core_axis_name="c", subcore_axis_name="s")

    @functools.partial(
        pl.kernel, mesh=mesh,
        out_type=jax.ShapeDtypeStruct((t, d), jnp.float32),
        scratch_types=[
            pltpu.VMEM((rows_per_w,), jnp.int32),
            pltpu.VMEM((_SC_CHUNK, d), jnp.float32),
            pltpu.SemaphoreType.DMA,
        ],
    )
    def k(x_hbm, idx_hbm, out_hbm, idx_v, rows_v, sem):
        wid = lax.axis_index("s") * _SC_CORES + lax.axis_index("c")
        base = wid * rows_per_w
        pltpu.sync_copy(idx_hbm.at[pl.ds(base, rows_per_w)], idx_v)
        for c in range(n_ch):
            pltpu.async_copy(
                x_hbm.at[idx_v.at[pl.ds(c * _SC_CHUNK, _SC_CHUNK)]],
                rows_v, sem).wait()
            pltpu.sync_copy(rows_v, out_hbm.at[pl.ds(base + c * _SC_CHUNK,
                                                     _SC_CHUNK)])

    return k(x, perm)


def _sc_scatter(ys, perm3):
    t, d = ys.shape
    rows_per_w = t // _SC_WORKERS
    n_ch = rows_per_w // _SC_CHUNK
    mesh = plsc.VectorSubcoreMesh(core_axis_name="c", subcore_axis_name="s")

    @functools.partial(
        pl.kernel, mesh=mesh,
        out_type=jax.ShapeDtypeStruct((t, d), jnp.float32),
        scratch_types=[
            pltpu.VMEM((n_ch, _SC_CHUNK), jnp.int32),
            pltpu.VMEM((_SC_CHUNK, d), jnp.float32),
            pltpu.SemaphoreType.DMA,
        ],
    )
    def k(ys_hbm, idx_hbm, out_hbm, idx_v, rows_v, sem):
        wid = lax.axis_index("s") * _SC_CORES + lax.axis_index("c")
        base = wid * rows_per_w
        pltpu.sync_copy(idx_hbm.at[wid], idx_v)
        for c in range(n_ch):
            pltpu.sync_copy(ys_hbm.at[pl.ds(base + c * _SC_CHUNK, _SC_CHUNK)],
                            rows_v)
            pltpu.async_copy(rows_v, out_hbm.at[idx_v.at[c]], sem).wait()

    return k(ys, perm3)


def _grouped_mlp(xs, gate2, w1, b1r, w2, b2r, g_meta, t_meta, lo, hi, first):
    t, d = xs.shape
    e, _, f = w1.shape
    num_slots = g_meta.shape[0]

    def body(g_ref, t_ref, lo_ref, hi_ref, first_ref,
             xs_ref, w1_ref, b1_ref, w2_ref, b2_ref, gate_ref, out_ref):
        i = pl.program_id(0)
        row0 = t_ref[i] * _TM
        ridx = row0 + lax.broadcasted_iota(jnp.int32, (_TM, 1), 0)
        mask = (ridx >= lo_ref[i]) & (ridx < hi_ref[i])
        h = jnp.dot(xs_ref[...], w1_ref[0],
                    preferred_element_type=jnp.float32) + b1_ref[0]
        h = jax.nn.gelu(h)
        y = jnp.dot(h, w2_ref[0], preferred_element_type=jnp.float32) + b2_ref[0]
        scale = jnp.where(mask, gate_ref[...], 0.0)
        contrib = y * scale

        @pl.when(first_ref[i] != 0)
        def _init():
            out_ref[...] = contrib

        @pl.when(first_ref[i] == 0)
        def _accum():
            out_ref[...] += contrib

    grid_spec = pltpu.PrefetchScalarGridSpec(
        num_scalar_prefetch=5,
        grid=(num_slots,),
        in_specs=[
            pl.BlockSpec((_TM, d), lambda i, g, tt, lo_, hi_, fr: (tt[i], 0)),
            pl.BlockSpec((1, d, f), lambda i, g, tt, lo_, hi_, fr: (g[i], 0, 0)),
            pl.BlockSpec((1, 1, f), lambda i, g, tt, lo_, hi_, fr: (g[i], 0, 0)),
            pl.BlockSpec((1, f, d), lambda i, g, tt, lo_, hi_, fr: (g[i], 0, 0)),
            pl.BlockSpec((1, 1, d), lambda i, g, tt, lo_, hi_, fr: (g[i], 0, 0)),
            pl.BlockSpec((_TM, 1), lambda i, g, tt, lo_, hi_, fr: (tt[i], 0)),
        ],
        out_specs=pl.BlockSpec((_TM, d), lambda i, g, tt, lo_, hi_, fr: (tt[i], 0)),
    )
    return pl.pallas_call(
        body,
        grid_spec=grid_spec,
        out_shape=jax.ShapeDtypeStruct((t, d), jnp.float32),
        compiler_params=pltpu.CompilerParams(
            dimension_semantics=("arbitrary",)),
    )(g_meta, t_meta, lo, hi, first, xs, w1, b1r, w2, b2r, gate2)


def kernel(hidden_states, W_router, W1, b1, W2, b2):
    s, b, d = hidden_states.shape
    e, _, f = W1.shape
    x = hidden_states.reshape(-1, d)
    t = x.shape[0]
    nt = t // _TM
    num_slots = nt + e

    gate, eid = _router(x, W_router)
    eid = eid.reshape(-1)
    gate = gate.reshape(-1)

    perm, g_meta, t_meta, lo, hi, first = _route_metadata(
        eid, e, _TM, num_slots)

    xs = _sc_gather(x, perm)
    gate_s = jnp.take(gate, perm).reshape(t, 1)

    b1r = b1.reshape(e, 1, f)
    b2r = b2.reshape(e, 1, d)
    ys = _grouped_mlp(xs, gate_s, W1, b1r, W2, b2r,
                      g_meta, t_meta, lo, hi, first)

    rows_per_w = t // _SC_WORKERS
    n_ch = rows_per_w // _SC_CHUNK
    out = _sc_scatter(ys, perm.reshape(_SC_WORKERS, n_ch, _SC_CHUNK))
    return out.reshape(s, b, d)
